# baseline (speedup 1.0000x reference)
import dataclasses
import functools

import jax
import jax.numpy as jnp
from jax import lax
from jax.experimental import pallas as pl
from jax.experimental.pallas import tpu as pltpu
from jax.experimental.pallas import tpu_sc as plsc

N = 10000
F = 128
NPAD = 10240
NC = 2
NS = 16
NW = NC * NS
CH = 128
RPT = NPAD // NS
RB = 2000

_mesh = plsc.VectorSubcoreMesh(core_axis_name="c", subcore_axis_name="s")


_sc_params = pltpu.CompilerParams()
if "needs_layout_passes" in pltpu.CompilerParams.__dataclass_fields__:
    _sc_params = dataclasses.replace(_sc_params, needs_layout_passes=False)


@functools.lru_cache(maxsize=None)
def _deg_kernel(nch):
    @functools.partial(
        pl.kernel,
        mesh=_mesh,
        compiler_params=_sc_params,
        out_type=jax.ShapeDtypeStruct((NC, NPAD), jnp.float32),
        scratch_types=[
            pltpu.VMEM((nch, CH), jnp.int32),
            pltpu.VMEM((NPAD,), jnp.float32),
            pltpu.VMEM((RPT,), jnp.float32),
            pltpu.VMEM((RPT,), jnp.float32),
            pltpu.VMEM_SHARED((NS, NPAD), jnp.float32),
        ],
    )
    def deg_k(dst_hbm, out_hbm, dst_v, hist, accv, tmpv, sh):
        c = lax.axis_index("c")
        s = lax.axis_index("s")
        t = c * NS + s
        pltpu.sync_copy(dst_hbm.at[t], dst_v)
        zz = jnp.zeros((16,), jnp.float32)
        ones = jnp.ones((16,), jnp.float32)

        @pl.loop(0, NPAD, step=16)
        def _(i):
            hist[pl.ds(i, 16)] = zz

        @pl.loop(0, nch)
        def _(k):
            @pl.loop(0, CH, step=16)
            def _(g):
                plsc.addupdate_scatter(hist, [dst_v.at[k][pl.ds(g, 16)]], ones)

        pltpu.sync_copy(hist, sh.at[s])
        plsc.subcore_barrier()

        @pl.loop(0, RPT, step=16)
        def _(i):
            accv[pl.ds(i, 16)] = zz

        for j in range(NS):
            pltpu.sync_copy(sh.at[j].at[pl.ds(s * RPT, RPT)], tmpv)

            @pl.loop(0, RPT, step=16)
            def _(i):
                accv[pl.ds(i, 16)] = accv[pl.ds(i, 16)] + tmpv[pl.ds(i, 16)]

        pltpu.sync_copy(accv, out_hbm.at[c].at[pl.ds(s * RPT, RPT)])

    return deg_k


@functools.lru_cache(maxsize=None)
def _conv_kernel(q0, q1):
    qm = max(q0, q1)

    @functools.partial(
        pl.kernel,
        mesh=_mesh,
        out_type=jax.ShapeDtypeStruct((NC, NPAD, F), jnp.float32),
        scratch_types=[
            pltpu.VMEM((qm, CH), jnp.int32),
            pltpu.VMEM((qm, CH), jnp.int32),
            pltpu.VMEM((CH, F), jnp.float32),
            pltpu.VMEM_SHARED((NPAD, F), jnp.float32),
            pltpu.SemaphoreType.DMA,
        ],
    )
    def conv_k(h_hbm, src_hbm, dst_hbm, zeros_hbm, out_hbm,
               src_v, dst_v, buf, acc_sh, gsa):
        c = lax.axis_index("c")
        s = lax.axis_index("s")
        pltpu.sync_copy(zeros_hbm, acc_sh.at[pl.ds(s * RPT, RPT)])
        plsc.subcore_barrier()

        def work(base, q):
            pltpu.sync_copy(src_hbm.at[pl.ds(base, q)], src_v.at[pl.ds(0, q)])
            pltpu.sync_copy(dst_hbm.at[pl.ds(base, q)], dst_v.at[pl.ds(0, q)])

            @pl.loop(0, q)
            def _(k):
                pltpu.async_copy(h_hbm.at[src_v.at[k]], buf, gsa).wait()
                pltpu.sync_copy(buf, acc_sh.at[dst_v.at[k]], add=True)

        @pl.when(c == 0)
        def _():
            work(s * q0, q0)

        @pl.when(c == 1)
        def _():
            work(NS * q0 + s * q1, q1)

        plsc.subcore_barrier()
        pltpu.sync_copy(acc_sh.at[pl.ds(s * RPT, RPT)],
                        out_hbm.at[c].at[pl.ds(s * RPT, RPT)])

    return conv_k


def _row_spec(cols):
    return pl.BlockSpec((RB, cols), lambda i: (i, 0))


def _full_spec(shape):
    return pl.BlockSpec(shape, lambda i: (0, 0))


def _scale_matmul(x, W, dc0, dc1):
    def body(x_ref, w_ref, d0_ref, d1_ref, o_ref):
        dinv = lax.rsqrt(d0_ref[...] + d1_ref[...] + 1.0)
        h = jnp.dot(x_ref[...], w_ref[...], preferred_element_type=jnp.float32)
        o_ref[...] = h * dinv

    return pl.pallas_call(
        body,
        grid=(N // RB,),
        in_specs=[_row_spec(F), _full_spec((F, F)), _row_spec(1), _row_spec(1)],
        out_specs=_row_spec(F),
        out_shape=jax.ShapeDtypeStruct((N, F), jnp.float32),
    )(x, W, dc0, dc1)


def _mid_stage(a0, a1, hp, dc0, dc1, b_g1, w_l1, b_l1, w_g2):
    def body(a0r, a1r, hpr, d0r, d1r, bgr, wlr, blr, wgr, o_ref):
        dinv = lax.rsqrt(d0r[...] + d1r[...] + 1.0)
        z = jnp.maximum(dinv * (a0r[...] + a1r[...] + hpr[...]) + bgr[...], 0.0)
        y = jnp.dot(z, wlr[...], preferred_element_type=jnp.float32) + blr[...]
        h2 = jnp.dot(y, wgr[...], preferred_element_type=jnp.float32)
        o_ref[...] = h2 * dinv

    return pl.pallas_call(
        body,
        grid=(N // RB,),
        in_specs=[_row_spec(F), _row_spec(F), _row_spec(F), _row_spec(1),
                  _row_spec(1), _full_spec((1, F)), _full_spec((F, F)),
                  _full_spec((1, F)), _full_spec((F, F))],
        out_specs=_row_spec(F),
        out_shape=jax.ShapeDtypeStruct((N, F), jnp.float32),
    )(a0, a1, hp, dc0, dc1, b_g1, w_l1, b_l1, w_g2)


def _final_stage(a0, a1, hp, dc0, dc1, b_g2, w_l2, b_l2):
    def body(a0r, a1r, hpr, d0r, d1r, bgr, wlr, blr, o_ref):
        dinv = lax.rsqrt(d0r[...] + d1r[...] + 1.0)
        z = jnp.maximum(dinv * (a0r[...] + a1r[...] + hpr[...]) + bgr[...], 0.0)
        o_ref[...] = jnp.dot(z, wlr[...], preferred_element_type=jnp.float32) + blr[...]

    return pl.pallas_call(
        body,
        grid=(N // RB,),
        in_specs=[_row_spec(F), _row_spec(F), _row_spec(F), _row_spec(1),
                  _row_spec(1), _full_spec((1, F)), _full_spec((F, F)),
                  _full_spec((1, F))],
        out_specs=_row_spec(F),
        out_shape=jax.ShapeDtypeStruct((N, F), jnp.float32),
    )(a0, a1, hp, dc0, dc1, b_g2, w_l2, b_l2)


def kernel(x, edge_index, gc1_W, gc1_b, lin1_W, lin1_b,
           gc2_W, gc2_b, lin2_W, lin2_b):
    E = edge_index.shape[1]
    ech = (E + CH - 1) // CH
    qt = (ech + NS - 1) // NS
    q0 = max(8, ((qt // 3) + 7) // 8 * 8)
    q1 = ((qt - q0) + 7) // 8 * 8
    tq = NS * (q0 + q1)
    nchd = (ech + NW - 1) // NW
    L = max(tq, nchd * NW) * CH

    src_p = jnp.concatenate([edge_index[0], jnp.zeros((L - E,), jnp.int32)])
    dst_p = jnp.concatenate([edge_index[1], jnp.full((L - E,), N, jnp.int32)])
    srcc = src_p[:tq * CH].reshape(tq, CH)
    dstc = dst_p[:tq * CH].reshape(tq, CH)
    dst3 = dst_p[:nchd * NW * CH].reshape(NW, nchd, CH)

    zerosF = jnp.zeros((RPT, F), jnp.float32)

    degp = _deg_kernel(nchd)(dst3)
    dc0 = degp[0, :N].reshape(N, 1)
    dc1 = degp[1, :N].reshape(N, 1)

    b_g1 = gc1_b.reshape(1, F)
    b_l1 = lin1_b.reshape(1, F)
    b_g2 = gc2_b.reshape(1, F)
    b_l2 = lin2_b.reshape(1, F)

    h1p = _scale_matmul(x, gc1_W, dc0, dc1)
    acc1 = _conv_kernel(q0, q1)(h1p, srcc, dstc, zerosF)
    h2p = _mid_stage(acc1[0, :N], acc1[1, :N], h1p, dc0, dc1,
                     b_g1, lin1_W, b_l1, gc2_W)
    acc2 = _conv_kernel(q0, q1)(h2p, srcc, dstc, zerosF)
    return _final_stage(acc2[0, :N], acc2[1, :N], h2p, dc0, dc1,
                        b_g2, lin2_W, b_l2)


# speedup vs baseline: 1.1694x; 1.1694x over previous
import dataclasses
import functools

import jax
import jax.numpy as jnp
from jax import lax
from jax.experimental import pallas as pl
from jax.experimental.pallas import tpu as pltpu
from jax.experimental.pallas import tpu_sc as plsc

N = 10000
F = 128
NPAD = 10240
NC = 2
NS = 16
NW = NC * NS
CH = 128
RPT = NPAD // NS
RB = 2000

_mesh = plsc.VectorSubcoreMesh(core_axis_name="c", subcore_axis_name="s")


_sc_params = pltpu.CompilerParams()
if "needs_layout_passes" in pltpu.CompilerParams.__dataclass_fields__:
    _sc_params = dataclasses.replace(_sc_params, needs_layout_passes=False)


@functools.lru_cache(maxsize=None)
def _deg_kernel(nch):
    @functools.partial(
        pl.kernel,
        mesh=_mesh,
        compiler_params=_sc_params,
        out_type=jax.ShapeDtypeStruct((NC, NPAD), jnp.float32),
        scratch_types=[
            pltpu.VMEM((nch, CH), jnp.int32),
            pltpu.VMEM((NPAD,), jnp.float32),
            pltpu.VMEM((RPT,), jnp.float32),
            pltpu.VMEM((RPT,), jnp.float32),
            pltpu.VMEM_SHARED((NS, NPAD), jnp.float32),
        ],
    )
    def deg_k(dst_hbm, out_hbm, dst_v, hist, accv, tmpv, sh):
        c = lax.axis_index("c")
        s = lax.axis_index("s")
        t = c * NS + s
        pltpu.sync_copy(dst_hbm.at[t], dst_v)
        zz = jnp.zeros((16,), jnp.float32)
        ones = jnp.ones((16,), jnp.float32)

        @pl.loop(0, NPAD, step=16)
        def _(i):
            hist[pl.ds(i, 16)] = zz

        @pl.loop(0, nch)
        def _(k):
            @pl.loop(0, CH, step=16)
            def _(g):
                plsc.addupdate_scatter(hist, [dst_v.at[k][pl.ds(g, 16)]], ones)

        pltpu.sync_copy(hist, sh.at[s])
        plsc.subcore_barrier()

        @pl.loop(0, RPT, step=16)
        def _(i):
            accv[pl.ds(i, 16)] = zz

        for j in range(NS):
            pltpu.sync_copy(sh.at[j].at[pl.ds(s * RPT, RPT)], tmpv)

            @pl.loop(0, RPT, step=16)
            def _(i):
                accv[pl.ds(i, 16)] = accv[pl.ds(i, 16)] + tmpv[pl.ds(i, 16)]

        pltpu.sync_copy(accv, out_hbm.at[c].at[pl.ds(s * RPT, RPT)])

    return deg_k


@functools.lru_cache(maxsize=None)
def _conv_kernel(q0, q1):
    qm = max(q0, q1)

    @functools.partial(
        pl.kernel,
        mesh=_mesh,
        out_type=jax.ShapeDtypeStruct((NC, NPAD, F), jnp.float32),
        scratch_types=[
            pltpu.VMEM((qm, CH), jnp.int32),
            pltpu.VMEM((qm, CH), jnp.int32),
            pltpu.VMEM((CH, F), jnp.float32),
            pltpu.VMEM_SHARED((NPAD, F), jnp.float32),
            pltpu.SemaphoreType.DMA,
        ],
    )
    def conv_k(h_hbm, src_hbm, dst_hbm, zeros_hbm, out_hbm,
               src_v, dst_v, buf, acc_sh, gsa):
        c = lax.axis_index("c")
        s = lax.axis_index("s")
        pltpu.sync_copy(zeros_hbm, acc_sh.at[pl.ds(s * RPT, RPT)])
        plsc.subcore_barrier()

        def work(base, q):
            pltpu.sync_copy(src_hbm.at[pl.ds(base, q)], src_v.at[pl.ds(0, q)])
            pltpu.sync_copy(dst_hbm.at[pl.ds(base, q)], dst_v.at[pl.ds(0, q)])

            @pl.loop(0, q)
            def _(k):
                pltpu.async_copy(h_hbm.at[src_v.at[k]], buf, gsa).wait()
                pltpu.sync_copy(buf, acc_sh.at[dst_v.at[k]], add=True)

        @pl.when(c == 0)
        def _():
            work(s * q0, q0)

        @pl.when(c == 1)
        def _():
            work(NS * q0 + s * q1, q1)

        plsc.subcore_barrier()
        pltpu.sync_copy(acc_sh.at[pl.ds(s * RPT, RPT)],
                        out_hbm.at[c].at[pl.ds(s * RPT, RPT)])

    return conv_k


def _row_spec(cols):
    return pl.BlockSpec((RB, cols), lambda i: (i, 0))


def _full_spec(shape):
    return pl.BlockSpec(shape, lambda i: (0, 0))


def _scale_matmul(x, W, dc0, dc1):
    def body(x_ref, w_ref, d0_ref, d1_ref, o_ref):
        dinv = lax.rsqrt(d0_ref[...] + d1_ref[...] + 1.0)
        h = jnp.dot(x_ref[...], w_ref[...], preferred_element_type=jnp.float32)
        o_ref[...] = h * dinv

    return pl.pallas_call(
        body,
        grid=(N // RB,),
        in_specs=[_row_spec(F), _full_spec((F, F)), _row_spec(1), _row_spec(1)],
        out_specs=_row_spec(F),
        out_shape=jax.ShapeDtypeStruct((N, F), jnp.float32),
    )(x, W, dc0, dc1)


def _mid_stage(a0, a1, hp, dc0, dc1, b_g1, w_l1, b_l1, w_g2):
    def body(a0r, a1r, hpr, d0r, d1r, bgr, wlr, blr, wgr, o_ref):
        dinv = lax.rsqrt(d0r[...] + d1r[...] + 1.0)
        z = jnp.maximum(dinv * (a0r[...] + a1r[...] + hpr[...]) + bgr[...], 0.0)
        y = jnp.dot(z, wlr[...], preferred_element_type=jnp.float32) + blr[...]
        h2 = jnp.dot(y, wgr[...], preferred_element_type=jnp.float32)
        o_ref[...] = h2 * dinv

    return pl.pallas_call(
        body,
        grid=(N // RB,),
        in_specs=[_row_spec(F), _row_spec(F), _row_spec(F), _row_spec(1),
                  _row_spec(1), _full_spec((1, F)), _full_spec((F, F)),
                  _full_spec((1, F)), _full_spec((F, F))],
        out_specs=_row_spec(F),
        out_shape=jax.ShapeDtypeStruct((N, F), jnp.float32),
    )(a0, a1, hp, dc0, dc1, b_g1, w_l1, b_l1, w_g2)


def _final_stage(a0, a1, hp, dc0, dc1, b_g2, w_l2, b_l2):
    def body(a0r, a1r, hpr, d0r, d1r, bgr, wlr, blr, o_ref):
        dinv = lax.rsqrt(d0r[...] + d1r[...] + 1.0)
        z = jnp.maximum(dinv * (a0r[...] + a1r[...] + hpr[...]) + bgr[...], 0.0)
        o_ref[...] = jnp.dot(z, wlr[...], preferred_element_type=jnp.float32) + blr[...]

    return pl.pallas_call(
        body,
        grid=(N // RB,),
        in_specs=[_row_spec(F), _row_spec(F), _row_spec(F), _row_spec(1),
                  _row_spec(1), _full_spec((1, F)), _full_spec((F, F)),
                  _full_spec((1, F))],
        out_specs=_row_spec(F),
        out_shape=jax.ShapeDtypeStruct((N, F), jnp.float32),
    )(a0, a1, hp, dc0, dc1, b_g2, w_l2, b_l2)


def kernel(x, edge_index, gc1_W, gc1_b, lin1_W, lin1_b,
           gc2_W, gc2_b, lin2_W, lin2_b):
    E = edge_index.shape[1]
    ech = (E + CH - 1) // CH
    qt = (ech + NS - 1) // NS
    q1 = max(8, ((qt // 3) + 7) // 8 * 8)
    q0 = ((qt - q1) + 7) // 8 * 8
    tq = NS * (q0 + q1)
    nchd = (ech + NW - 1) // NW
    L = max(tq, nchd * NW) * CH

    src_p = jnp.concatenate([edge_index[0], jnp.zeros((L - E,), jnp.int32)])
    dst_p = jnp.concatenate([edge_index[1], jnp.full((L - E,), N, jnp.int32)])
    srcc = src_p[:tq * CH].reshape(tq, CH)
    dstc = dst_p[:tq * CH].reshape(tq, CH)
    dst3 = dst_p[:nchd * NW * CH].reshape(NW, nchd, CH)

    zerosF = jnp.zeros((RPT, F), jnp.float32)

    degp = _deg_kernel(nchd)(dst3)
    dc0 = degp[0, :N].reshape(N, 1)
    dc1 = degp[1, :N].reshape(N, 1)

    b_g1 = gc1_b.reshape(1, F)
    b_l1 = lin1_b.reshape(1, F)
    b_g2 = gc2_b.reshape(1, F)
    b_l2 = lin2_b.reshape(1, F)

    h1p = _scale_matmul(x, gc1_W, dc0, dc1)
    acc1 = _conv_kernel(q0, q1)(h1p, srcc, dstc, zerosF)
    h2p = _mid_stage(acc1[0, :N], acc1[1, :N], h1p, dc0, dc1,
                     b_g1, lin1_W, b_l1, gc2_W)
    acc2 = _conv_kernel(q0, q1)(h2p, srcc, dstc, zerosF)
    return _final_stage(acc2[0, :N], acc2[1, :N], h2p, dc0, dc1,
                        b_g2, lin2_W, b_l2)


# speedup vs baseline: 1.6862x; 1.4420x over previous
import dataclasses
import functools

import jax
import jax.numpy as jnp
from jax import lax
from jax.experimental import pallas as pl
from jax.experimental.pallas import tpu as pltpu
from jax.experimental.pallas import tpu_sc as plsc

N = 10000
F = 128
NPAD = 10240
NC = 2
NS = 16
NW = NC * NS
CH = 128
RPT = NPAD // NS
RB = 2000

_mesh = plsc.VectorSubcoreMesh(core_axis_name="c", subcore_axis_name="s")


_sc_params = pltpu.CompilerParams()
if "needs_layout_passes" in pltpu.CompilerParams.__dataclass_fields__:
    _sc_params = dataclasses.replace(_sc_params, needs_layout_passes=False)


@functools.lru_cache(maxsize=None)
def _deg_kernel(nch):
    @functools.partial(
        pl.kernel,
        mesh=_mesh,
        compiler_params=_sc_params,
        out_type=jax.ShapeDtypeStruct((NC, NPAD), jnp.float32),
        scratch_types=[
            pltpu.VMEM((nch, CH), jnp.int32),
            pltpu.VMEM((NPAD,), jnp.float32),
            pltpu.VMEM((RPT,), jnp.float32),
            pltpu.VMEM((RPT,), jnp.float32),
            pltpu.VMEM_SHARED((NS, NPAD), jnp.float32),
        ],
    )
    def deg_k(dst_hbm, out_hbm, dst_v, hist, accv, tmpv, sh):
        c = lax.axis_index("c")
        s = lax.axis_index("s")
        t = c * NS + s
        pltpu.sync_copy(dst_hbm.at[t], dst_v)
        zz = jnp.zeros((16,), jnp.float32)
        ones = jnp.ones((16,), jnp.float32)

        @pl.loop(0, NPAD, step=16)
        def _(i):
            hist[pl.ds(i, 16)] = zz

        @pl.loop(0, nch)
        def _(k):
            @pl.loop(0, CH, step=16)
            def _(g):
                plsc.addupdate_scatter(hist, [dst_v.at[k][pl.ds(g, 16)]], ones)

        pltpu.sync_copy(hist, sh.at[s])
        plsc.subcore_barrier()

        @pl.loop(0, RPT, step=16)
        def _(i):
            accv[pl.ds(i, 16)] = zz

        for j in range(NS):
            pltpu.sync_copy(sh.at[j].at[pl.ds(s * RPT, RPT)], tmpv)

            @pl.loop(0, RPT, step=16)
            def _(i):
                accv[pl.ds(i, 16)] = accv[pl.ds(i, 16)] + tmpv[pl.ds(i, 16)]

        pltpu.sync_copy(accv, out_hbm.at[c].at[pl.ds(s * RPT, RPT)])

    return deg_k


@functools.lru_cache(maxsize=None)
def _conv_kernel(nch):
    @functools.partial(
        pl.kernel,
        mesh=_mesh,
        out_type=jax.ShapeDtypeStruct((NC, NPAD, F), jnp.float32),
        scratch_types=[
            pltpu.VMEM((nch, CH), jnp.int32),
            pltpu.VMEM((nch, CH), jnp.int32),
            pltpu.VMEM((CH, F), jnp.float32),
            pltpu.VMEM_SHARED((NPAD, F), jnp.float32),
            pltpu.SemaphoreType.DMA,
        ],
    )
    def conv_k(h_hbm, src_hbm, dst_hbm, zeros_hbm, out_hbm,
               src_v, dst_v, buf, acc_sh, gsa):
        c = lax.axis_index("c")
        s = lax.axis_index("s")
        t = c * NS + s
        pltpu.sync_copy(src_hbm.at[t], src_v)
        pltpu.sync_copy(dst_hbm.at[t], dst_v)
        pltpu.sync_copy(zeros_hbm, acc_sh.at[pl.ds(s * RPT, RPT)])
        plsc.subcore_barrier()

        @pl.loop(0, nch)
        def _(k):
            pltpu.async_copy(h_hbm.at[src_v.at[k]], buf, gsa).wait()
            pltpu.sync_copy(buf, acc_sh.at[dst_v.at[k]], add=True)

        plsc.subcore_barrier()
        pltpu.sync_copy(acc_sh.at[pl.ds(s * RPT, RPT)],
                        out_hbm.at[c].at[pl.ds(s * RPT, RPT)])

    return conv_k


def _row_spec(cols):
    return pl.BlockSpec((RB, cols), lambda i: (i, 0))


def _full_spec(shape):
    return pl.BlockSpec(shape, lambda i: (0, 0))


def _scale_matmul(x, W, dc0, dc1):
    def body(x_ref, w_ref, d0_ref, d1_ref, o_ref):
        dinv = lax.rsqrt(d0_ref[...] + d1_ref[...] + 1.0)
        h = jnp.dot(x_ref[...], w_ref[...], preferred_element_type=jnp.float32)
        o_ref[...] = h * dinv

    return pl.pallas_call(
        body,
        grid=(N // RB,),
        in_specs=[_row_spec(F), _full_spec((F, F)), _row_spec(1), _row_spec(1)],
        out_specs=_row_spec(F),
        out_shape=jax.ShapeDtypeStruct((N, F), jnp.float32),
    )(x, W, dc0, dc1)


def _mid_stage(a0, a1, hp, dc0, dc1, b_g1, w_l1, b_l1, w_g2):
    def body(a0r, a1r, hpr, d0r, d1r, bgr, wlr, blr, wgr, o_ref):
        dinv = lax.rsqrt(d0r[...] + d1r[...] + 1.0)
        z = jnp.maximum(dinv * (a0r[...] + a1r[...] + hpr[...]) + bgr[...], 0.0)
        y = jnp.dot(z, wlr[...], preferred_element_type=jnp.float32) + blr[...]
        h2 = jnp.dot(y, wgr[...], preferred_element_type=jnp.float32)
        o_ref[...] = h2 * dinv

    return pl.pallas_call(
        body,
        grid=(N // RB,),
        in_specs=[_row_spec(F), _row_spec(F), _row_spec(F), _row_spec(1),
                  _row_spec(1), _full_spec((1, F)), _full_spec((F, F)),
                  _full_spec((1, F)), _full_spec((F, F))],
        out_specs=_row_spec(F),
        out_shape=jax.ShapeDtypeStruct((N, F), jnp.float32),
    )(a0, a1, hp, dc0, dc1, b_g1, w_l1, b_l1, w_g2)


def _final_stage(a0, a1, hp, dc0, dc1, b_g2, w_l2, b_l2):
    def body(a0r, a1r, hpr, d0r, d1r, bgr, wlr, blr, o_ref):
        dinv = lax.rsqrt(d0r[...] + d1r[...] + 1.0)
        z = jnp.maximum(dinv * (a0r[...] + a1r[...] + hpr[...]) + bgr[...], 0.0)
        o_ref[...] = jnp.dot(z, wlr[...], preferred_element_type=jnp.float32) + blr[...]

    return pl.pallas_call(
        body,
        grid=(N // RB,),
        in_specs=[_row_spec(F), _row_spec(F), _row_spec(F), _row_spec(1),
                  _row_spec(1), _full_spec((1, F)), _full_spec((F, F)),
                  _full_spec((1, F))],
        out_specs=_row_spec(F),
        out_shape=jax.ShapeDtypeStruct((N, F), jnp.float32),
    )(a0, a1, hp, dc0, dc1, b_g2, w_l2, b_l2)


def kernel(x, edge_index, gc1_W, gc1_b, lin1_W, lin1_b,
           gc2_W, gc2_b, lin2_W, lin2_b):
    E = edge_index.shape[1]
    grp = NW * CH
    nch = (E + grp - 1) // grp
    L = nch * grp

    src3 = jnp.concatenate(
        [edge_index[0], jnp.zeros((L - E,), jnp.int32)]).reshape(NW, nch, CH)
    dst3 = jnp.concatenate(
        [edge_index[1], jnp.full((L - E,), N, jnp.int32)]).reshape(NW, nch, CH)

    zerosF = jnp.zeros((RPT, F), jnp.float32)

    degp = _deg_kernel(nch)(dst3)
    dc0 = degp[0, :N].reshape(N, 1)
    dc1 = degp[1, :N].reshape(N, 1)

    b_g1 = gc1_b.reshape(1, F)
    b_l1 = lin1_b.reshape(1, F)
    b_g2 = gc2_b.reshape(1, F)
    b_l2 = lin2_b.reshape(1, F)

    h1p = _scale_matmul(x, gc1_W, dc0, dc1)
    acc1 = _conv_kernel(nch)(h1p, src3, dst3, zerosF)
    h2p = _mid_stage(acc1[0, :N], acc1[1, :N], h1p, dc0, dc1,
                     b_g1, lin1_W, b_l1, gc2_W)
    acc2 = _conv_kernel(nch)(h2p, src3, dst3, zerosF)
    return _final_stage(acc2[0, :N], acc2[1, :N], h2p, dc0, dc1,
                        b_g2, lin2_W, b_l2)
